# baseline (speedup 1.0000x reference)
import jax
import jax.numpy as jnp
from jax import lax
from jax.experimental import pallas as pl
from jax.experimental.pallas import tpu as pltpu


def _selection_mats():
    col = jnp.arange(256)[None, :]
    b, q = col // 128, col % 128
    k = jnp.arange(148)[:, None]
    s1m = ((k == 2 * q + b - 2) & (q < 76)).astype(jnp.bfloat16)
    row = jnp.arange(256)[:, None]
    a, p = row // 128, row % 128
    r = jnp.arange(148)[None, :]
    r1m = ((r == 2 * p + a - 2) & (p < 76)).astype(jnp.bfloat16)
    k2 = jnp.arange(128)[:, None]
    col = jnp.arange(256)[None, :]
    b, l = col // 128, col % 128
    s2m = ((k2 == 2 * (l % 64) + b - 2) & (k2 < 74)).astype(jnp.bfloat16)
    row = jnp.arange(96)[:, None]
    s, ar = row // 48, row % 48
    a, r2 = ar // 24, ar % 24
    j = jnp.arange(74)[None, :]
    rsm = ((j == 4 * r2 + a - 2 + 2 * s) & (r2 < 21)).astype(jnp.bfloat16)
    return s1m, r1m, s2m, rsm


def _w1_perm():
    idx = [t * 16 + a * 8 + b * 4 + ci
           for ci in range(3) for t in range(9) for a in range(2) for b in range(2)]
    return jnp.array(idx, jnp.int32)


def _fwd1_body(x_ref, w1_ref, s1_ref, s1m_ref, r1m_ref, s2m_ref, rsm_ref,
               o_ref, xs_ref):
    s1m, r1m = s1m_ref[...], r1m_ref[...]
    for ci in range(3):
        xc = x_ref[0, ci].astype(jnp.bfloat16)
        u = jnp.dot(xc, s1m, preferred_element_type=jnp.float32)
        pc = jnp.dot(r1m, u.astype(jnp.bfloat16),
                     preferred_element_type=jnp.float32).astype(jnp.bfloat16)
        for t in range(9):
            r, c = t // 3, t % 3
            for a in range(2):
                for b in range(2):
                    kk = ci * 36 + t * 4 + a * 2 + b
                    if b == 1 and c > 0:
                        src = jnp.pad(pc[a * 128 + r:a * 128 + r + 74,
                                         128 + c:256], ((0, 0), (0, c)))
                    else:
                        src = pc[a * 128 + r:a * 128 + r + 74,
                                 b * 128 + c:b * 128 + c + 128]
                    xs_ref[kk] = src
    z = jnp.einsum('mk,kpq->mpq', w1_ref[...], xs_ref[...],
                   preferred_element_type=jnp.float32)
    z = jnp.maximum(z + s1_ref[...][:, :, None], 0.0)
    y1 = jnp.maximum(jnp.maximum(z[:16], z[16:32]),
                     jnp.maximum(z[32:48], z[48:64])).astype(jnp.bfloat16)
    s2m, rsm = s2m_ref[...], rsm_ref[...]
    lane = lax.broadcasted_iota(jnp.int32, (21, 256), 1) % 128
    for ci in range(16):
        v = jnp.dot(y1[ci], s2m,
                    preferred_element_type=jnp.float32).astype(jnp.bfloat16)
        g = jnp.dot(rsm, v,
                    preferred_element_type=jnp.float32).astype(jnp.bfloat16)
        for a in range(2):
            pk = jnp.where(lane < 64, g[a * 24:a * 24 + 21],
                           g[48 + a * 24:48 + a * 24 + 21])
            o_ref[0, a * 32 + ci] = pk[:, 0:128]
            o_ref[0, a * 32 + 16 + ci] = pk[:, 128:256]


def _fwd2_body(p_ref, w2_ref, s2_ref, wfc_ref, o_ref, xs_ref):
    x4 = p_ref[0]
    d = jnp.concatenate([x4[:, 0:19], x4[:, 1:20], x4[:, 2:21]], axis=2)
    for t in range(9):
        r, c = t // 3, t % 3
        off = r * 64 + c
        xs_ref[t * 64:(t + 1) * 64] = d[:, :, off:off + 128]
    z = jnp.einsum('mk,krl->mrl', w2_ref[...], xs_ref[...],
                   preferred_element_type=jnp.float32)
    z = jnp.maximum(z + s2_ref[...][:, :, None], 0.0)
    y2 = jnp.maximum(jnp.maximum(z[:32], z[32:64]),
                     jnp.maximum(z[64:96], z[96:128]))
    o_ref[0, 0] = jnp.sum(wfc_ref[...] * y2[None], axis=(1, 2, 3))


@jax.jit
def kernel(x, w1, s1, w2, s2, wfc, bfc):
    n = x.shape[0]
    s1m, r1m, s2m, rsm = _selection_mats()
    w1p = w1[:, _w1_perm()]

    packed = pl.pallas_call(
        _fwd1_body,
        out_shape=jax.ShapeDtypeStruct((n, 64, 21, 128), jnp.bfloat16),
        grid=(n,),
        in_specs=[pl.BlockSpec((1, 3, 148, 148), lambda i: (i, 0, 0, 0)),
                  pl.BlockSpec((64, 108), lambda i: (0, 0)),
                  pl.BlockSpec((64, 1), lambda i: (0, 0)),
                  pl.BlockSpec((148, 256), lambda i: (0, 0)),
                  pl.BlockSpec((256, 148), lambda i: (0, 0)),
                  pl.BlockSpec((128, 256), lambda i: (0, 0)),
                  pl.BlockSpec((96, 74), lambda i: (0, 0))],
        out_specs=pl.BlockSpec((1, 64, 21, 128), lambda i: (i, 0, 0, 0)),
        scratch_shapes=[pltpu.VMEM((108, 74, 128), jnp.bfloat16)],
        compiler_params=pltpu.CompilerParams(
            dimension_semantics=("parallel",),
            vmem_limit_bytes=64 * 1024 * 1024),
    )(x, w1p, s1, s1m, r1m, s2m, rsm)

    logits = pl.pallas_call(
        _fwd2_body,
        out_shape=jax.ShapeDtypeStruct((n, 1, 6), jnp.float32),
        grid=(n,),
        in_specs=[pl.BlockSpec((1, 64, 21, 128), lambda i: (i, 0, 0, 0)),
                  pl.BlockSpec((128, 576), lambda i: (0, 0)),
                  pl.BlockSpec((128, 1), lambda i: (0, 0)),
                  pl.BlockSpec((6, 32, 19, 128), lambda i: (0, 0, 0, 0))],
        out_specs=pl.BlockSpec((1, 1, 6), lambda i: (i, 0, 0)),
        scratch_shapes=[pltpu.VMEM((576, 19, 128), jnp.bfloat16)],
        compiler_params=pltpu.CompilerParams(
            dimension_semantics=("parallel",),
            vmem_limit_bytes=64 * 1024 * 1024),
    )(packed, w2, s2, wfc.reshape(6, 32, 19, 128))

    return logits.reshape(n, 6) + bfc[None, :]


# speedup vs baseline: 1.1445x; 1.1445x over previous
import jax
import jax.numpy as jnp
from jax import lax
from jax.experimental import pallas as pl
from jax.experimental.pallas import tpu as pltpu


def _selection_mats():
    col = jnp.arange(256)[None, :]
    b, q = col // 128, col % 128
    k = jnp.arange(148)[:, None]
    s1m = ((k == 2 * q + b - 2) & (q < 76)).astype(jnp.bfloat16)
    row = jnp.arange(256)[:, None]
    a, p = row // 128, row % 128
    r = jnp.arange(148)[None, :]
    r1m = ((r == 2 * p + a - 2) & (p < 76)).astype(jnp.bfloat16)
    k2 = jnp.arange(128)[:, None]
    col = jnp.arange(256)[None, :]
    b, l = col // 128, col % 128
    s2m = ((k2 == 2 * (l % 64) + b - 2) & (k2 < 74)).astype(jnp.bfloat16)
    row = jnp.arange(96)[:, None]
    s, ar = row // 48, row % 48
    a, r2 = ar // 24, ar % 24
    j = jnp.arange(74)[None, :]
    rsm = ((j == 4 * r2 + a - 2 + 2 * s) & (r2 < 21)).astype(jnp.bfloat16)
    return s1m, r1m, s2m, rsm


def _w1_perm():
    idx = [t * 16 + a * 8 + b * 4 + ci
           for ci in range(3) for t in range(9) for a in range(2) for b in range(2)]
    return jnp.array(idx, jnp.int32)


def _fwd1_body(x_ref, w1_ref, s1_ref, s1m_ref, r1m_ref, s2m_ref, rsm_ref,
               o_ref):
    s1m, r1m = s1m_ref[...], r1m_ref[...]
    parts = []
    for ci in range(3):
        xc = x_ref[0, ci].astype(jnp.bfloat16)
        u = jnp.dot(xc, s1m, preferred_element_type=jnp.float32)
        pc = jnp.dot(r1m, u.astype(jnp.bfloat16),
                     preferred_element_type=jnp.float32).astype(jnp.bfloat16)
        for t in range(9):
            r, c = t // 3, t % 3
            for a in range(2):
                for b in range(2):
                    if b == 1 and c > 0:
                        src = jnp.pad(pc[a * 128 + r:a * 128 + r + 74,
                                         128 + c:256], ((0, 0), (0, c)))
                    else:
                        src = pc[a * 128 + r:a * 128 + r + 74,
                                 b * 128 + c:b * 128 + c + 128]
                    parts.append(src)
    xs = jnp.stack(parts, axis=0)
    z = jnp.einsum('mk,kpq->mpq', w1_ref[...], xs,
                   preferred_element_type=jnp.float32)
    z = jnp.maximum(z + s1_ref[:, 0:1, :], 0.0)
    y1 = jnp.maximum(jnp.maximum(z[:16], z[16:32]),
                     jnp.maximum(z[32:48], z[48:64])).astype(jnp.bfloat16)
    s2m, rsm = s2m_ref[...], rsm_ref[...]
    lane = lax.broadcasted_iota(jnp.int32, (21, 256), 1) % 128
    for ci in range(16):
        v = jnp.dot(y1[ci], s2m,
                    preferred_element_type=jnp.float32).astype(jnp.bfloat16)
        g = jnp.dot(rsm, v,
                    preferred_element_type=jnp.float32).astype(jnp.bfloat16)
        for a in range(2):
            pk = jnp.where(lane < 64, g[a * 24:a * 24 + 21],
                           g[48 + a * 24:48 + a * 24 + 21])
            o_ref[0, a * 32 + ci] = pk[:, 0:128]
            o_ref[0, a * 32 + 16 + ci] = pk[:, 128:256]


def _fwd2_body(p_ref, w2_ref, s2_ref, wfc_ref, o_ref):
    x4 = p_ref[0]
    d = jnp.concatenate([x4[:, 0:19], x4[:, 1:20], x4[:, 2:21]], axis=2)
    xs = jnp.concatenate(
        [d[:, :, (t // 3) * 64 + (t % 3):(t // 3) * 64 + (t % 3) + 128]
         for t in range(9)], axis=0)
    z = jnp.einsum('mk,krl->mrl', w2_ref[...], xs,
                   preferred_element_type=jnp.float32)
    z = jnp.maximum(z + s2_ref[:, 0:1, :], 0.0)
    y2 = jnp.maximum(jnp.maximum(z[:32], z[32:64]),
                     jnp.maximum(z[64:96], z[96:128]))
    o_ref[0, 0] = jnp.sum(wfc_ref[...] * y2[None], axis=(1, 2, 3))


@jax.jit
def kernel(x, w1, s1, w2, s2, wfc, bfc):
    n = x.shape[0]
    s1m, r1m, s2m, rsm = _selection_mats()
    w1p = w1[:, _w1_perm()]
    s1b = jnp.broadcast_to(s1.reshape(64, 1, 1), (64, 8, 128))
    s2b = jnp.broadcast_to(s2.reshape(128, 1, 1), (128, 8, 128))

    packed = pl.pallas_call(
        _fwd1_body,
        out_shape=jax.ShapeDtypeStruct((n, 64, 21, 128), jnp.bfloat16),
        grid=(n,),
        in_specs=[pl.BlockSpec((1, 3, 148, 148), lambda i: (i, 0, 0, 0)),
                  pl.BlockSpec((64, 108), lambda i: (0, 0)),
                  pl.BlockSpec((64, 8, 128), lambda i: (0, 0, 0)),
                  pl.BlockSpec((148, 256), lambda i: (0, 0)),
                  pl.BlockSpec((256, 148), lambda i: (0, 0)),
                  pl.BlockSpec((128, 256), lambda i: (0, 0)),
                  pl.BlockSpec((96, 74), lambda i: (0, 0))],
        out_specs=pl.BlockSpec((1, 64, 21, 128), lambda i: (i, 0, 0, 0)),
        compiler_params=pltpu.CompilerParams(
            dimension_semantics=("parallel",),
            vmem_limit_bytes=64 * 1024 * 1024),
    )(x, w1p, s1b, s1m, r1m, s2m, rsm)

    logits = pl.pallas_call(
        _fwd2_body,
        out_shape=jax.ShapeDtypeStruct((n, 1, 6), jnp.float32),
        grid=(n,),
        in_specs=[pl.BlockSpec((1, 64, 21, 128), lambda i: (i, 0, 0, 0)),
                  pl.BlockSpec((128, 576), lambda i: (0, 0)),
                  pl.BlockSpec((128, 8, 128), lambda i: (0, 0, 0)),
                  pl.BlockSpec((6, 32, 19, 128), lambda i: (0, 0, 0, 0))],
        out_specs=pl.BlockSpec((1, 1, 6), lambda i: (i, 0, 0)),
        compiler_params=pltpu.CompilerParams(
            dimension_semantics=("parallel",),
            vmem_limit_bytes=64 * 1024 * 1024),
    )(packed, w2, s2b, wfc.reshape(6, 32, 19, 128))

    return logits.reshape(n, 6) + bfc[None, :]
